# Initial kernel scaffold; baseline (speedup 1.0000x reference)
import jax
import jax.numpy as jnp
from jax.experimental import pallas as pl


def kernel(x, edge_attr, params, edge_index, batch, batch_size):
    raise NotImplementedError("write your pallas kernel here")


# speedup vs baseline: 9.4727x; 9.4727x over previous
import functools
import math

import jax
import jax.numpy as jnp
from jax import lax
from jax.experimental import pallas as pl
from jax.experimental.pallas import tpu as pltpu
from jax.experimental.pallas import tpu_sc as plsc

_NC = 2
_NS = 16
_NW = _NC * _NS
_CH = 128
_ZR = 128


def _cdiv(a, b):
    return (a + b - 1) // b


@functools.lru_cache(maxsize=None)
def _make_spmm(n, h, nchunk, npad):
    mesh = plsc.VectorSubcoreMesh(core_axis_name="c", subcore_axis_name="s")
    rps = npad // _NS

    @functools.partial(
        pl.kernel,
        mesh=mesh,
        out_type=jax.ShapeDtypeStruct((_NC, npad, h), jnp.float32),
        scratch_types=[
            pltpu.VMEM((nchunk, _CH), jnp.int32),
            pltpu.VMEM((nchunk, _CH), jnp.int32),
            pltpu.VMEM((nchunk, _CH), jnp.float32),
            pltpu.VMEM((_CH, h), jnp.float32),
            pltpu.VMEM_SHARED((npad, h), jnp.float32),
            pltpu.SemaphoreType.DMA,
        ],
    )
    def spmm(y_hbm, src_hbm, dst_hbm, ew_hbm, out_hbm,
             src_v, dst_v, ew_v, rows_v, acc_sh, gsem):
        c = lax.axis_index("c")
        s = lax.axis_index("s")
        w = s * _NC + c

        zv = jnp.zeros((16,), jnp.float32)

        def zero_rows(i, carry):
            for k in range(h // 16):
                rows_v[i, pl.ds(k * 16, 16)] = zv
            return carry

        lax.fori_loop(0, _CH, zero_rows, 0)
        for z in range(rps // _ZR):
            pltpu.sync_copy(rows_v, acc_sh.at[pl.ds(s * rps + z * _ZR, _ZR)])
        plsc.subcore_barrier()

        pltpu.sync_copy(src_hbm.at[pl.ds(w * nchunk, nchunk)], src_v)
        pltpu.sync_copy(dst_hbm.at[pl.ds(w * nchunk, nchunk)], dst_v)
        pltpu.sync_copy(ew_hbm.at[pl.ds(w * nchunk, nchunk)], ew_v)

        def chunk_body(g, carry):
            pltpu.async_copy(y_hbm.at[src_v.at[g]], rows_v, gsem).wait()

            def scale_body(i2, c2):
                ev = ew_v[g, pl.ds(i2 * 16, 16)]
                for j in range(16):
                    vs = jnp.full((16,), ev[j], jnp.float32)
                    r = i2 * 16 + j
                    for k in range(h // 16):
                        rows_v[r, pl.ds(k * 16, 16)] = (
                            rows_v[r, pl.ds(k * 16, 16)] * vs)
                return c2

            lax.fori_loop(0, _CH // 16, scale_body, 0)
            pltpu.sync_copy(rows_v, acc_sh.at[dst_v.at[g]], add=True)
            return carry

        lax.fori_loop(0, nchunk, chunk_body, 0)
        plsc.subcore_barrier()
        pltpu.sync_copy(acc_sh.at[pl.ds(s * rps, rps)],
                        out_hbm.at[c, pl.ds(s * rps, rps)])

    return spmm


@functools.lru_cache(maxsize=None)
def _make_deg(nchunk, npad):
    mesh = plsc.VectorSubcoreMesh(core_axis_name="c", subcore_axis_name="s")
    rps = npad // _NS

    @functools.partial(
        pl.kernel,
        mesh=mesh,
        out_type=jax.ShapeDtypeStruct((_NC * npad,), jnp.float32),
        scratch_types=[
            pltpu.VMEM((nchunk, _CH), jnp.int32),
            pltpu.VMEM((nchunk, _CH), jnp.float32),
            pltpu.VMEM((rps,), jnp.float32),
            pltpu.VMEM_SHARED((npad,), jnp.float32),
            pltpu.SemaphoreType.DMA,
        ],
    )
    def degk(dst_hbm, ew_hbm, out_hbm, dst_v, ew_v, stage_v, deg_sh, sem):
        c = lax.axis_index("c")
        s = lax.axis_index("s")
        w = s * _NC + c

        zv = jnp.zeros((16,), jnp.float32)

        def zero_stage(i, carry):
            stage_v[pl.ds(i * 16, 16)] = zv
            return carry

        lax.fori_loop(0, rps // 16, zero_stage, 0)
        pltpu.sync_copy(stage_v, deg_sh.at[pl.ds(s * rps, rps)])
        plsc.subcore_barrier()

        pltpu.sync_copy(dst_hbm.at[pl.ds(w * nchunk, nchunk)], dst_v)
        pltpu.sync_copy(ew_hbm.at[pl.ds(w * nchunk, nchunk)], ew_v)

        def chunk_body(g, carry):
            pltpu.sync_copy(ew_v.at[g], deg_sh.at[dst_v.at[g]], add=True)
            return carry

        lax.fori_loop(0, nchunk, chunk_body, 0)
        plsc.subcore_barrier()
        pltpu.sync_copy(deg_sh.at[pl.ds(s * rps, rps)],
                        out_hbm.at[pl.ds(c * npad + s * rps, rps)])

    return degk


_CHR = 1000


def _dinv_tc(dp):
    n = dp.shape[1]

    def body(d_ref, o_ref):
        d = d_ref[0:1, :] + d_ref[1:2, :] + 1.0
        o_ref[...] = lax.rsqrt(d)

    return pl.pallas_call(
        body,
        out_shape=jax.ShapeDtypeStruct((1, n), jnp.float32),
    )(dp)


def _mm_scale_tc(x, w, dinv):
    n, f = x.shape
    h = w.shape[1]

    def body(x_ref, w_ref, d_ref, y_ref):
        y_ref[...] = d_ref[...] * jnp.dot(
            x_ref[...], w_ref[...], preferred_element_type=jnp.float32)

    return pl.pallas_call(
        body,
        grid=(n // _CHR,),
        in_specs=[
            pl.BlockSpec((_CHR, f), lambda i: (i, 0)),
            pl.BlockSpec((f, h), lambda i: (0, 0)),
            pl.BlockSpec((_CHR, 1), lambda i: (i, 0)),
        ],
        out_specs=pl.BlockSpec((_CHR, h), lambda i: (i, 0)),
        out_shape=jax.ShapeDtypeStruct((n, h), jnp.float32),
    )(x, w, dinv)


def _step_tc(accp, y, b, dinv, wnext):
    n, h = y.shape
    npad = accp.shape[1]

    def body(a_ref, y_ref, b_ref, d_ref, w_ref, h_ref, y2_ref):
        a = a_ref[0] + a_ref[1]
        d = d_ref[...]
        hh = jnp.maximum(d * (a + y_ref[...]) + b_ref[...], 0.0)
        h_ref[...] = hh
        y2_ref[...] = d * jnp.dot(hh, w_ref[...],
                                  preferred_element_type=jnp.float32)

    return pl.pallas_call(
        body,
        grid=(n // _CHR,),
        in_specs=[
            pl.BlockSpec((_NC, _CHR, h), lambda i: (0, i, 0)),
            pl.BlockSpec((_CHR, h), lambda i: (i, 0)),
            pl.BlockSpec((1, h), lambda i: (0, 0)),
            pl.BlockSpec((_CHR, 1), lambda i: (i, 0)),
            pl.BlockSpec((h, h), lambda i: (0, 0)),
        ],
        out_specs=[
            pl.BlockSpec((_CHR, h), lambda i: (i, 0)),
            pl.BlockSpec((_CHR, h), lambda i: (i, 0)),
        ],
        out_shape=[
            jax.ShapeDtypeStruct((n, h), jnp.float32),
            jax.ShapeDtypeStruct((n, h), jnp.float32),
        ],
    )(accp, y, b, dinv, wnext)


def _big_step_tc(accp, y, b, dinv, h0, wja, wjb, bj, batch_row, nseg, w0n):
    n, h = y.shape
    last = w0n is None

    def body(a_ref, y_ref, b_ref, d_ref, h0_ref, wa_ref, wb_ref, bj_ref,
             bt_ref, *rest):
        if last:
            p_ref, = rest[-1:]
        else:
            w0_ref = rest[0]
            p_ref, y2_ref = rest[1], rest[2]
        i = pl.program_id(0)
        a = a_ref[0] + a_ref[1]
        d = d_ref[...]
        h1 = jnp.maximum(d * (a + y_ref[...]) + b_ref[...], 0.0)
        hb = jnp.maximum(
            jnp.dot(h0_ref[...], wa_ref[...], preferred_element_type=jnp.float32)
            + jnp.dot(h1, wb_ref[...], preferred_element_type=jnp.float32)
            + bj_ref[...], 0.0)
        seg = lax.broadcasted_iota(jnp.int32, (_CHR, nseg), 1)
        onehot = (bt_ref[...] == seg).astype(jnp.float32)

        @pl.when(i == 0)
        def _():
            p_ref[...] = jnp.zeros_like(p_ref)

        p_ref[...] += lax.dot_general(
            onehot, hb, (((0,), (0,)), ((), ())),
            preferred_element_type=jnp.float32)
        if not last:
            y2_ref[...] = d * jnp.dot(hb, w0_ref[...],
                                      preferred_element_type=jnp.float32)

    in_specs = [
        pl.BlockSpec((_NC, _CHR, h), lambda i: (0, i, 0)),
        pl.BlockSpec((_CHR, h), lambda i: (i, 0)),
        pl.BlockSpec((1, h), lambda i: (0, 0)),
        pl.BlockSpec((_CHR, 1), lambda i: (i, 0)),
        pl.BlockSpec((_CHR, h), lambda i: (i, 0)),
        pl.BlockSpec((h, h), lambda i: (0, 0)),
        pl.BlockSpec((h, h), lambda i: (0, 0)),
        pl.BlockSpec((1, h), lambda i: (0, 0)),
        pl.BlockSpec((_CHR, 1), lambda i: (i, 0)),
    ]
    out_specs = [pl.BlockSpec((nseg, h), lambda i: (0, 0))]
    out_shape = [jax.ShapeDtypeStruct((nseg, h), jnp.float32)]
    args = [accp, y, b, dinv, h0, wja, wjb, bj, batch_row]
    if not last:
        in_specs.append(pl.BlockSpec((h, h), lambda i: (0, 0)))
        out_specs.append(pl.BlockSpec((_CHR, h), lambda i: (i, 0)))
        out_shape.append(jax.ShapeDtypeStruct((n, h), jnp.float32))
        args.append(w0n)

    return pl.pallas_call(
        body,
        grid=(n // _CHR,),
        in_specs=in_specs,
        out_specs=out_specs,
        out_shape=out_shape,
    )(*args)


def _head_tc(p1, p2, p3, bng, bnb, wl1, bl1, wl2, bl2, eps):
    nb = p1.shape[0]
    cdim = wl2.shape[1]
    c0 = 1.0 / math.sqrt(1.0 + eps)

    def body(p1_ref, p2_ref, p3_ref, g_ref, b_ref, w1_ref, b1_ref,
             w2_ref, b2_ref, o_ref):
        z = jnp.concatenate([p1_ref[...], p2_ref[...], p3_ref[...]], axis=1)
        z = g_ref[...] * z * c0 + b_ref[...]
        z = jnp.maximum(
            jnp.dot(z, w1_ref[...], preferred_element_type=jnp.float32)
            + b1_ref[...], 0.0)
        z = jnp.dot(z, w2_ref[...], preferred_element_type=jnp.float32) \
            + b2_ref[...]
        m = jnp.max(z, axis=-1, keepdims=True)
        e = jnp.exp(z - m)
        o_ref[...] = e / jnp.sum(e, axis=-1, keepdims=True)

    return pl.pallas_call(
        body,
        out_shape=jax.ShapeDtypeStruct((nb, cdim), jnp.float32),
    )(p1, p2, p3, bng, bnb, wl1, bl1, wl2, bl2)


def kernel(x, edge_attr, params, edge_index, batch, batch_size):
    n, f = x.shape
    e = edge_attr.shape[0]
    blocks = params['blocks']
    h = blocks[0]['W0'].shape[1]
    nseg = 64
    eps = 1e-5

    ew_per_w = e // _NW
    nchunk = _cdiv(_cdiv(ew_per_w, _CH), 8) * 8
    ewp = nchunk * _CH
    pad = ewp - ew_per_w
    npad = _NS * _ZR * _cdiv(n, _NS * _ZR)

    src = edge_index[0].reshape(_NW, ew_per_w)
    dst = edge_index[1].reshape(_NW, ew_per_w)
    eww = edge_attr.reshape(_NW, ew_per_w)
    if pad:
        pad_src = jnp.broadcast_to(
            jnp.arange(pad, dtype=jnp.int32) % jnp.int32(n), (_NW, pad))
        pad_dst = jnp.broadcast_to(
            jnp.int32(n) + jnp.arange(pad, dtype=jnp.int32)
            % jnp.int32(npad - n), (_NW, pad))
        src = jnp.concatenate([src, pad_src], axis=1)
        dst = jnp.concatenate([dst, pad_dst], axis=1)
        eww = jnp.concatenate(
            [eww, jnp.zeros((_NW, pad), jnp.float32)], axis=1)
    src_p = src.reshape(_NW * nchunk, _CH)
    dst_p = dst.reshape(_NW * nchunk, _CH)
    ew_p = eww.reshape(_NW * nchunk, _CH)

    spmm = _make_spmm(n, h, nchunk, npad)
    degk = _make_deg(nchunk, npad)

    deg_flat = degk(dst_p, ew_p)
    dp = deg_flat.reshape(_NC, npad)[:, :n]
    dinv = _dinv_tc(dp).reshape(n, 1)

    batch_row = batch.reshape(n, 1)

    y = _mm_scale_tc(x, blocks[0]['W0'], dinv)
    pooled = []
    for bi in range(3):
        p = blocks[bi]
        accp = spmm(y, src_p, dst_p, ew_p)
        h0, y = _step_tc(accp, y, p['b0'].reshape(1, h), dinv, p['W1'])
        accp = spmm(y, src_p, dst_p, ew_p)
        wja = p['Wj'][:h]
        wjb = p['Wj'][h:]
        w0n = blocks[bi + 1]['W0'] if bi < 2 else None
        outs = _big_step_tc(accp, y, p['b1'].reshape(1, h), dinv, h0,
                            wja, wjb, p['bj'].reshape(1, h),
                            batch_row, nseg, w0n)
        if bi < 2:
            pooled_bi, y = outs
        else:
            pooled_bi, = outs
        pooled.append(pooled_bi)

    out = _head_tc(pooled[0], pooled[1], pooled[2],
                   params['bn_g'].reshape(1, 3 * h),
                   params['bn_b'].reshape(1, 3 * h),
                   params['Wl1'], params['bl1'].reshape(1, h),
                   params['Wl2'], params['bl2'].reshape(1, 16),
                   eps)
    return out


# speedup vs baseline: 13.5995x; 1.4357x over previous
import functools
import math

import jax
import jax.numpy as jnp
from jax import lax
from jax.experimental import pallas as pl
from jax.experimental.pallas import tpu as pltpu
from jax.experimental.pallas import tpu_sc as plsc

_NC = 2
_NS = 16
_NW = _NC * _NS
_CH = 128
_ZR = 128
_NB = 4


def _cdiv(a, b):
    return (a + b - 1) // b


@functools.lru_cache(maxsize=None)
def _make_spmm(n, h, nchunk, npad):
    mesh = plsc.VectorSubcoreMesh(core_axis_name="c", subcore_axis_name="s")
    rps = npad // _NS
    qc = nchunk // 2
    assert qc % 8 == 0

    @functools.partial(
        pl.kernel,
        mesh=mesh,
        out_type=jax.ShapeDtypeStruct((_NC, npad, h), jnp.float32),
        scratch_types=(
            [pltpu.VMEM((qc, _CH), jnp.int32),
             pltpu.VMEM((qc, _CH), jnp.int32),
             pltpu.VMEM((qc, _CH), jnp.float32)]
            + [pltpu.VMEM((_CH, h), jnp.float32)] * 2
            + [pltpu.VMEM_SHARED((npad, h), jnp.float32)]
            + [pltpu.SemaphoreType.DMA] * 4
        ),
    )
    def spmm(y_hbm, src_hbm, dst_hbm, ew_hbm, out_hbm,
             src_v, dst_v, ew_v, *bufs_and_sems):
        full = list(bufs_and_sems[0:2])
        acc_sh = bufs_and_sems[2]
        gsem = list(bufs_and_sems[3:5])
        ssem = list(bufs_and_sems[5:7])
        c = lax.axis_index("c")
        s = lax.axis_index("s")
        w = s * _NC + c

        zv = jnp.zeros((16,), jnp.float32)

        def zero_rows(i, carry):
            for k in range(h // 16):
                full[0][i, pl.ds(k * 16, 16)] = zv
            return carry

        lax.fori_loop(0, _CH, zero_rows, 0)
        for z in range(rps // _ZR):
            pltpu.sync_copy(full[0], acc_sh.at[pl.ds(s * rps + z * _ZR, _ZR)])
        plsc.subcore_barrier()

        def load_idx(qi):
            base = w * nchunk + qi * qc
            pltpu.sync_copy(src_hbm.at[pl.ds(base, qc)], src_v)
            pltpu.sync_copy(dst_hbm.at[pl.ds(base, qc)], dst_v)
            pltpu.sync_copy(ew_hbm.at[pl.ds(base, qc)], ew_v)

        load_idx(0)
        pltpu.async_copy(y_hbm.at[src_v.at[0]], full[0], gsem[0])

        def scale(j, lr):
            def scale_body(i2, c2):
                ev = ew_v[lr, pl.ds(i2 * 16, 16)]
                for jj in range(16):
                    vs = jnp.full((16,), ev[jj], jnp.float32)
                    r = i2 * 16 + jj
                    for k in range(h // 16):
                        full[j][r, pl.ds(k * 16, 16)] = (
                            full[j][r, pl.ds(k * 16, 16)] * vs)
                return c2

            lax.fori_loop(0, _CH // 16, scale_body, 0)

        def super_body(big, carry):
            for j in range(2):
                g = big * 2 + j
                lr = lax.rem(g, qc)
                jp = 1 - j

                if j == 0:
                    @pl.when(jnp.logical_and(lr == 0, g > 0))
                    def _():
                        pltpu.make_async_copy(
                            full[1], acc_sh.at[dst_v.at[0]], ssem[1]).wait()
                        pltpu.make_async_copy(
                            full[0], acc_sh.at[dst_v.at[0]], ssem[0]).wait()
                        load_idx(g // qc)
                        pltpu.async_copy(
                            y_hbm.at[src_v.at[0]], full[0], gsem[0])

                pltpu.make_async_copy(
                    y_hbm.at[src_v.at[lr]], full[j], gsem[j]).wait()

                @pl.when(jnp.logical_and(g + 1 < nchunk,
                                         lax.rem(g + 1, qc) != 0))
                def _():
                    @pl.when(lr >= 1)
                    def _():
                        pltpu.make_async_copy(
                            full[jp], acc_sh.at[dst_v.at[lr]],
                            ssem[jp]).wait()

                    pltpu.async_copy(
                        y_hbm.at[src_v.at[lr + 1]], full[jp], gsem[jp])

                scale(j, lr)
                pltpu.async_copy(full[j], acc_sh.at[dst_v.at[lr]],
                                 ssem[j], add=True)
            return carry

        lax.fori_loop(0, nchunk // 2, super_body, 0)
        for j in range(2):
            pltpu.make_async_copy(
                full[j], acc_sh.at[dst_v.at[0]], ssem[j]).wait()
        plsc.subcore_barrier()
        pltpu.sync_copy(acc_sh.at[pl.ds(s * rps, rps)],
                        out_hbm.at[c, pl.ds(s * rps, rps)])

    return spmm


@functools.lru_cache(maxsize=None)
def _make_deg(nchunk, npad):
    mesh = plsc.VectorSubcoreMesh(core_axis_name="c", subcore_axis_name="s")
    rps = npad // _NS

    @functools.partial(
        pl.kernel,
        mesh=mesh,
        out_type=jax.ShapeDtypeStruct((_NC * npad,), jnp.float32),
        scratch_types=[
            pltpu.VMEM((nchunk, _CH), jnp.int32),
            pltpu.VMEM((nchunk, _CH), jnp.float32),
            pltpu.VMEM((rps,), jnp.float32),
            pltpu.VMEM_SHARED((npad,), jnp.float32),
            pltpu.SemaphoreType.DMA,
        ],
    )
    def degk(dst_hbm, ew_hbm, out_hbm, dst_v, ew_v, stage_v, deg_sh, sem):
        c = lax.axis_index("c")
        s = lax.axis_index("s")

        zv = jnp.zeros((16,), jnp.float32)

        def zero_stage(i, carry):
            stage_v[pl.ds(i * 16, 16)] = zv
            return carry

        lax.fori_loop(0, rps // 16, zero_stage, 0)
        pltpu.sync_copy(stage_v, deg_sh.at[pl.ds(s * rps, rps)])
        plsc.subcore_barrier()

        w = s * _NC + c
        pltpu.sync_copy(dst_hbm.at[pl.ds(w * nchunk, nchunk)], dst_v)
        pltpu.sync_copy(ew_hbm.at[pl.ds(w * nchunk, nchunk)], ew_v)

        def chunk_body(g, carry):
            pltpu.sync_copy(ew_v.at[g], deg_sh.at[dst_v.at[g]], add=True)
            return carry

        lax.fori_loop(0, nchunk, chunk_body, 0)
        plsc.subcore_barrier()
        pltpu.sync_copy(deg_sh.at[pl.ds(s * rps, rps)],
                        out_hbm.at[pl.ds(c * npad + s * rps, rps)])

    return degk


_CHR = 1000


def _dinv_tc(dp):
    n = dp.shape[1]

    def body(d_ref, o_ref):
        d = d_ref[0:1, :] + d_ref[1:2, :] + 1.0
        o_ref[...] = lax.rsqrt(d)

    return pl.pallas_call(
        body,
        out_shape=jax.ShapeDtypeStruct((1, n), jnp.float32),
    )(dp)


def _mm_scale_tc(x, w, dinv):
    n, f = x.shape
    h = w.shape[1]

    def body(x_ref, w_ref, d_ref, y_ref):
        y_ref[...] = d_ref[...] * jnp.dot(
            x_ref[...], w_ref[...], preferred_element_type=jnp.float32)

    return pl.pallas_call(
        body,
        grid=(n // _CHR,),
        in_specs=[
            pl.BlockSpec((_CHR, f), lambda i: (i, 0)),
            pl.BlockSpec((f, h), lambda i: (0, 0)),
            pl.BlockSpec((_CHR, 1), lambda i: (i, 0)),
        ],
        out_specs=pl.BlockSpec((_CHR, h), lambda i: (i, 0)),
        out_shape=jax.ShapeDtypeStruct((n, h), jnp.float32),
    )(x, w, dinv)


def _conv_step_tc(accp, y, hprev, dinv, batch_col, b, wja, wjb, bj, wn,
                  flag, nseg):
    npad = accp.shape[1]
    h2 = accp.shape[2]
    h = y.shape[1]
    n = y.shape[0]

    def body(a_ref, y_ref, hp_ref, d_ref, bt_ref, b_ref, wa_ref, wb_ref,
             bj_ref, wn_ref, fl_ref, h_ref, y2_ref, p_ref):
        i = pl.program_id(0)
        a = a_ref[0] + a_ref[1]
        d = d_ref[...]
        fl = fl_ref[...]
        h1 = jnp.maximum(d * (a + y_ref[...]) + b_ref[...], 0.0)
        hb = jnp.maximum(
            jnp.dot(hp_ref[...], wa_ref[...],
                    preferred_element_type=jnp.float32)
            + jnp.dot(h1, wb_ref[...], preferred_element_type=jnp.float32)
            + bj_ref[...], 0.0)
        sel = h1 + fl * (hb - h1)
        h_ref[...] = h1
        y2_ref[...] = d * jnp.dot(sel, wn_ref[...],
                                  preferred_element_type=jnp.float32)
        seg = lax.broadcasted_iota(jnp.int32, (_CHR, nseg), 1)
        onehot = (bt_ref[...] == seg).astype(jnp.float32)

        @pl.when(i == 0)
        def _():
            p_ref[...] = jnp.zeros_like(p_ref)

        p_ref[...] += lax.dot_general(
            onehot, fl * hb, (((0,), (0,)), ((), ())),
            preferred_element_type=jnp.float32)

    return pl.pallas_call(
        body,
        grid=(n // _CHR,),
        in_specs=[
            pl.BlockSpec((_NC, _CHR, h), lambda i: (0, i, 0)),
            pl.BlockSpec((_CHR, h), lambda i: (i, 0)),
            pl.BlockSpec((_CHR, h), lambda i: (i, 0)),
            pl.BlockSpec((_CHR, 1), lambda i: (i, 0)),
            pl.BlockSpec((_CHR, 1), lambda i: (i, 0)),
            pl.BlockSpec((1, h), lambda i: (0, 0)),
            pl.BlockSpec((h, h), lambda i: (0, 0)),
            pl.BlockSpec((h, h), lambda i: (0, 0)),
            pl.BlockSpec((1, h), lambda i: (0, 0)),
            pl.BlockSpec((h, h), lambda i: (0, 0)),
            pl.BlockSpec((1, 1), lambda i: (0, 0)),
        ],
        out_specs=[
            pl.BlockSpec((_CHR, h), lambda i: (i, 0)),
            pl.BlockSpec((_CHR, h), lambda i: (i, 0)),
            pl.BlockSpec((nseg, h), lambda i: (0, 0)),
        ],
        out_shape=[
            jax.ShapeDtypeStruct((n, h), jnp.float32),
            jax.ShapeDtypeStruct((n, h), jnp.float32),
            jax.ShapeDtypeStruct((nseg, h), jnp.float32),
        ],
    )(accp, y, hprev, dinv, batch_col, b, wja, wjb, bj, wn, flag)


def _head_tc(p1, p2, p3, bng, bnb, wl1, bl1, wl2, bl2, eps):
    nb = p1.shape[0]
    cdim = wl2.shape[1]
    c0 = 1.0 / math.sqrt(1.0 + eps)

    def body(p1_ref, p2_ref, p3_ref, g_ref, b_ref, w1_ref, b1_ref,
             w2_ref, b2_ref, o_ref):
        z = jnp.concatenate([p1_ref[...], p2_ref[...], p3_ref[...]], axis=1)
        z = g_ref[...] * z * c0 + b_ref[...]
        z = jnp.maximum(
            jnp.dot(z, w1_ref[...], preferred_element_type=jnp.float32)
            + b1_ref[...], 0.0)
        z = jnp.dot(z, w2_ref[...], preferred_element_type=jnp.float32) \
            + b2_ref[...]
        m = jnp.max(z, axis=-1, keepdims=True)
        e = jnp.exp(z - m)
        o_ref[...] = e / jnp.sum(e, axis=-1, keepdims=True)

    return pl.pallas_call(
        body,
        out_shape=jax.ShapeDtypeStruct((nb, cdim), jnp.float32),
    )(p1, p2, p3, bng, bnb, wl1, bl1, wl2, bl2)


def kernel(x, edge_attr, params, edge_index, batch, batch_size):
    n, f = x.shape
    e = edge_attr.shape[0]
    blocks = params['blocks']
    h = blocks[0]['W0'].shape[1]
    nseg = 64
    eps = 1e-5

    ew_per_w = e // _NW
    nchunk = _cdiv(_cdiv(ew_per_w, _CH), 8) * 8
    ewp = nchunk * _CH
    pad = ewp - ew_per_w
    npad = _NS * _ZR * _cdiv(n, _NS * _ZR)

    src = edge_index[0].reshape(_NW, ew_per_w)
    dst = edge_index[1].reshape(_NW, ew_per_w)
    eww = edge_attr.reshape(_NW, ew_per_w)
    if pad:
        pad_src = jnp.broadcast_to(
            jnp.arange(pad, dtype=jnp.int32) % jnp.int32(n), (_NW, pad))
        pad_dst = jnp.broadcast_to(
            jnp.int32(n) + jnp.arange(pad, dtype=jnp.int32)
            % jnp.int32(npad - n), (_NW, pad))
        src = jnp.concatenate([src, pad_src], axis=1)
        dst = jnp.concatenate([dst, pad_dst], axis=1)
        eww = jnp.concatenate(
            [eww, jnp.zeros((_NW, pad), jnp.float32)], axis=1)
    src_p = src.reshape(_NW * nchunk, _CH)
    dst_p = dst.reshape(_NW * nchunk, _CH)
    ew_p = eww.reshape(_NW * nchunk, _CH)

    spmm = _make_spmm(n, h, nchunk, npad)
    degk = _make_deg(nchunk, npad)

    deg_flat = degk(dst_p, ew_p)
    dp = deg_flat.reshape(_NC, npad)[:, :n]
    dinv = _dinv_tc(dp).reshape(n, 1)

    batch_col = batch.reshape(n, 1)

    zh = jnp.zeros((h, h), jnp.float32)
    zb = jnp.zeros((h,), jnp.float32)
    b_st, wja_st, wjb_st, bj_st, wn_st, fl_st = [], [], [], [], [], []
    for bi in range(3):
        p = blocks[bi]
        wn_last = blocks[bi + 1]['W0'] if bi < 2 else zh
        b_st += [p['b0'], p['b1']]
        wja_st += [zh, p['Wj'][:h]]
        wjb_st += [zh, p['Wj'][h:]]
        bj_st += [zb, p['bj']]
        wn_st += [p['W1'], wn_last]
        fl_st += [0.0, 1.0]
    b_st = jnp.stack(b_st).reshape(6, 1, h)
    wja_st = jnp.stack(wja_st)
    wjb_st = jnp.stack(wjb_st)
    bj_st = jnp.stack(bj_st).reshape(6, 1, h)
    wn_st = jnp.stack(wn_st)
    fl_st = jnp.asarray(fl_st, jnp.float32).reshape(6, 1, 1)

    y0 = _mm_scale_tc(x, blocks[0]['W0'], dinv)

    def take(stack, i):
        return lax.dynamic_index_in_dim(stack, i, 0, keepdims=False)

    def loop_body(i, carry):
        y, hprev, pooled = carry
        accp = spmm(y, src_p, dst_p, ew_p)
        h1, ynext, pooled_i = _conv_step_tc(
            accp, y, hprev, dinv, batch_col,
            take(b_st, i), take(wja_st, i), take(wjb_st, i),
            take(bj_st, i), take(wn_st, i), take(fl_st, i), nseg)
        pooled = lax.dynamic_update_slice(
            pooled, pooled_i[None], (i // 2, 0, 0))
        return (ynext, h1, pooled)

    ub = lax.optimization_barrier(jnp.int32(6))
    pooled0 = jnp.zeros((3, nseg, h), jnp.float32)
    _, _, pooled_all = lax.fori_loop(0, ub, loop_body, (y0, y0, pooled0))

    out = _head_tc(pooled_all[0], pooled_all[1], pooled_all[2],
                   params['bn_g'].reshape(1, 3 * h),
                   params['bn_b'].reshape(1, 3 * h),
                   params['Wl1'], params['bl1'].reshape(1, h),
                   params['Wl2'], params['bl2'].reshape(1, 16),
                   eps)
    return out


# speedup vs baseline: 13.7126x; 1.0083x over previous
import functools
import math

import jax
import jax.numpy as jnp
from jax import lax
from jax.experimental import pallas as pl
from jax.experimental.pallas import tpu as pltpu
from jax.experimental.pallas import tpu_sc as plsc

_NC = 2
_NS = 16
_NW = _NC * _NS
_CH = 128
_ZR = 128
_NB = 4


def _cdiv(a, b):
    return (a + b - 1) // b


@functools.lru_cache(maxsize=None)
def _make_spmm(n, h, nchunk, npad):
    mesh = plsc.VectorSubcoreMesh(core_axis_name="c", subcore_axis_name="s")
    rps = npad // _NS
    qc = nchunk // 2
    assert qc % 8 == 0

    @functools.partial(
        pl.kernel,
        mesh=mesh,
        out_type=jax.ShapeDtypeStruct((_NC, npad, h), jnp.float32),
        scratch_types=(
            [pltpu.VMEM((qc, _CH), jnp.int32),
             pltpu.VMEM((qc, _CH), jnp.int32),
             pltpu.VMEM((qc, _CH), jnp.float32)]
            + [pltpu.VMEM((_CH, h), jnp.float32)] * 2
            + [pltpu.VMEM_SHARED((npad, h), jnp.float32)]
            + [pltpu.SemaphoreType.DMA] * 4
        ),
    )
    def spmm(y_hbm, src_hbm, dst_hbm, ew_hbm, out_hbm,
             src_v, dst_v, ew_v, *bufs_and_sems):
        full = list(bufs_and_sems[0:2])
        acc_sh = bufs_and_sems[2]
        gsem = list(bufs_and_sems[3:5])
        ssem = list(bufs_and_sems[5:7])
        c = lax.axis_index("c")
        s = lax.axis_index("s")
        w = s * _NC + c

        zv = jnp.zeros((16,), jnp.float32)

        def zero_rows(i, carry):
            for k in range(h // 16):
                full[0][i, pl.ds(k * 16, 16)] = zv
            return carry

        lax.fori_loop(0, _CH, zero_rows, 0)
        for z in range(rps // _ZR):
            pltpu.async_copy(
                full[0], acc_sh.at[pl.ds(s * rps + z * _ZR, _ZR)], gsem[0])
        for z in range(rps // _ZR):
            pltpu.make_async_copy(
                full[0], acc_sh.at[pl.ds(s * rps + z * _ZR, _ZR)],
                gsem[0]).wait()
        plsc.subcore_barrier()

        def load_idx(qi):
            base = w * nchunk + qi * qc
            pltpu.sync_copy(src_hbm.at[pl.ds(base, qc)], src_v)
            pltpu.sync_copy(dst_hbm.at[pl.ds(base, qc)], dst_v)
            pltpu.sync_copy(ew_hbm.at[pl.ds(base, qc)], ew_v)

        load_idx(0)
        pltpu.async_copy(y_hbm.at[src_v.at[0]], full[0], gsem[0])

        def scale(j, lr):
            def scale_body(i2, c2):
                ev = ew_v[lr, pl.ds(i2 * 16, 16)]
                for jj in range(16):
                    vs = jnp.full((16,), ev[jj], jnp.float32)
                    r = i2 * 16 + jj
                    for k in range(h // 16):
                        full[j][r, pl.ds(k * 16, 16)] = (
                            full[j][r, pl.ds(k * 16, 16)] * vs)
                return c2

            lax.fori_loop(0, _CH // 16, scale_body, 0)

        def super_body(big, carry):
            for j in range(2):
                g = big * 2 + j
                lr = lax.rem(g, qc)
                jp = 1 - j

                if j == 0:
                    @pl.when(jnp.logical_and(lr == 0, g > 0))
                    def _():
                        pltpu.make_async_copy(
                            full[1], acc_sh.at[dst_v.at[0]], ssem[1]).wait()
                        pltpu.make_async_copy(
                            full[0], acc_sh.at[dst_v.at[0]], ssem[0]).wait()
                        load_idx(g // qc)
                        pltpu.async_copy(
                            y_hbm.at[src_v.at[0]], full[0], gsem[0])

                pltpu.make_async_copy(
                    y_hbm.at[src_v.at[lr]], full[j], gsem[j]).wait()

                @pl.when(jnp.logical_and(g + 1 < nchunk,
                                         lax.rem(g + 1, qc) != 0))
                def _():
                    @pl.when(lr >= 1)
                    def _():
                        pltpu.make_async_copy(
                            full[jp], acc_sh.at[dst_v.at[lr]],
                            ssem[jp]).wait()

                    pltpu.async_copy(
                        y_hbm.at[src_v.at[lr + 1]], full[jp], gsem[jp])

                scale(j, lr)
                pltpu.async_copy(full[j], acc_sh.at[dst_v.at[lr]],
                                 ssem[j], add=True)
            return carry

        lax.fori_loop(0, nchunk // 2, super_body, 0)
        for j in range(2):
            pltpu.make_async_copy(
                full[j], acc_sh.at[dst_v.at[0]], ssem[j]).wait()
        plsc.subcore_barrier()
        pltpu.sync_copy(acc_sh.at[pl.ds(s * rps, rps)],
                        out_hbm.at[c, pl.ds(s * rps, rps)])

    return spmm


@functools.lru_cache(maxsize=None)
def _make_deg(nchunk, npad):
    mesh = plsc.VectorSubcoreMesh(core_axis_name="c", subcore_axis_name="s")
    rps = npad // _NS

    @functools.partial(
        pl.kernel,
        mesh=mesh,
        out_type=jax.ShapeDtypeStruct((_NC * npad,), jnp.float32),
        scratch_types=[
            pltpu.VMEM((nchunk, _CH), jnp.int32),
            pltpu.VMEM((nchunk, _CH), jnp.float32),
            pltpu.VMEM((rps,), jnp.float32),
            pltpu.VMEM_SHARED((npad,), jnp.float32),
            pltpu.SemaphoreType.DMA,
        ],
    )
    def degk(dst_hbm, ew_hbm, out_hbm, dst_v, ew_v, stage_v, deg_sh, sem):
        c = lax.axis_index("c")
        s = lax.axis_index("s")

        zv = jnp.zeros((16,), jnp.float32)

        def zero_stage(i, carry):
            stage_v[pl.ds(i * 16, 16)] = zv
            return carry

        lax.fori_loop(0, rps // 16, zero_stage, 0)
        pltpu.sync_copy(stage_v, deg_sh.at[pl.ds(s * rps, rps)])
        plsc.subcore_barrier()

        w = s * _NC + c
        pltpu.sync_copy(dst_hbm.at[pl.ds(w * nchunk, nchunk)], dst_v)
        pltpu.sync_copy(ew_hbm.at[pl.ds(w * nchunk, nchunk)], ew_v)

        def chunk_body(g, carry):
            pltpu.async_copy(ew_v.at[g], deg_sh.at[dst_v.at[g]], sem,
                             add=True)
            return carry

        lax.fori_loop(0, nchunk, chunk_body, 0)

        def drain_body(g, carry):
            pltpu.make_async_copy(
                ew_v.at[0], deg_sh.at[dst_v.at[0]], sem).wait()
            return carry

        lax.fori_loop(0, nchunk, drain_body, 0)
        plsc.subcore_barrier()
        pltpu.sync_copy(deg_sh.at[pl.ds(s * rps, rps)],
                        out_hbm.at[pl.ds(c * npad + s * rps, rps)])

    return degk


_CHR = 1000


def _dinv_tc(dp):
    n = dp.shape[1]

    def body(d_ref, o_ref):
        d = d_ref[0:1, :] + d_ref[1:2, :] + 1.0
        o_ref[...] = lax.rsqrt(d)

    return pl.pallas_call(
        body,
        out_shape=jax.ShapeDtypeStruct((1, n), jnp.float32),
    )(dp)


def _mm_scale_tc(x, w, dinv):
    n, f = x.shape
    h = w.shape[1]

    def body(x_ref, w_ref, d_ref, y_ref):
        y_ref[...] = d_ref[...] * jnp.dot(
            x_ref[...], w_ref[...], preferred_element_type=jnp.float32)

    return pl.pallas_call(
        body,
        grid=(n // _CHR,),
        in_specs=[
            pl.BlockSpec((_CHR, f), lambda i: (i, 0)),
            pl.BlockSpec((f, h), lambda i: (0, 0)),
            pl.BlockSpec((_CHR, 1), lambda i: (i, 0)),
        ],
        out_specs=pl.BlockSpec((_CHR, h), lambda i: (i, 0)),
        out_shape=jax.ShapeDtypeStruct((n, h), jnp.float32),
    )(x, w, dinv)


def _conv_step_tc(accp, y, hprev, dinv, batch_col, b, wja, wjb, bj, wn,
                  flag, nseg):
    npad = accp.shape[1]
    h2 = accp.shape[2]
    h = y.shape[1]
    n = y.shape[0]

    def body(a_ref, y_ref, hp_ref, d_ref, bt_ref, b_ref, wa_ref, wb_ref,
             bj_ref, wn_ref, fl_ref, h_ref, y2_ref, p_ref):
        i = pl.program_id(0)
        a = a_ref[0] + a_ref[1]
        d = d_ref[...]
        fl = fl_ref[...]
        h1 = jnp.maximum(d * (a + y_ref[...]) + b_ref[...], 0.0)
        hb = jnp.maximum(
            jnp.dot(hp_ref[...], wa_ref[...],
                    preferred_element_type=jnp.float32)
            + jnp.dot(h1, wb_ref[...], preferred_element_type=jnp.float32)
            + bj_ref[...], 0.0)
        sel = h1 + fl * (hb - h1)
        h_ref[...] = h1
        y2_ref[...] = d * jnp.dot(sel, wn_ref[...],
                                  preferred_element_type=jnp.float32)
        seg = lax.broadcasted_iota(jnp.int32, (_CHR, nseg), 1)
        onehot = (bt_ref[...] == seg).astype(jnp.float32)

        @pl.when(i == 0)
        def _():
            p_ref[...] = jnp.zeros_like(p_ref)

        p_ref[...] += lax.dot_general(
            onehot, fl * hb, (((0,), (0,)), ((), ())),
            preferred_element_type=jnp.float32)

    return pl.pallas_call(
        body,
        grid=(n // _CHR,),
        in_specs=[
            pl.BlockSpec((_NC, _CHR, h), lambda i: (0, i, 0)),
            pl.BlockSpec((_CHR, h), lambda i: (i, 0)),
            pl.BlockSpec((_CHR, h), lambda i: (i, 0)),
            pl.BlockSpec((_CHR, 1), lambda i: (i, 0)),
            pl.BlockSpec((_CHR, 1), lambda i: (i, 0)),
            pl.BlockSpec((1, h), lambda i: (0, 0)),
            pl.BlockSpec((h, h), lambda i: (0, 0)),
            pl.BlockSpec((h, h), lambda i: (0, 0)),
            pl.BlockSpec((1, h), lambda i: (0, 0)),
            pl.BlockSpec((h, h), lambda i: (0, 0)),
            pl.BlockSpec((1, 1), lambda i: (0, 0)),
        ],
        out_specs=[
            pl.BlockSpec((_CHR, h), lambda i: (i, 0)),
            pl.BlockSpec((_CHR, h), lambda i: (i, 0)),
            pl.BlockSpec((nseg, h), lambda i: (0, 0)),
        ],
        out_shape=[
            jax.ShapeDtypeStruct((n, h), jnp.float32),
            jax.ShapeDtypeStruct((n, h), jnp.float32),
            jax.ShapeDtypeStruct((nseg, h), jnp.float32),
        ],
    )(accp, y, hprev, dinv, batch_col, b, wja, wjb, bj, wn, flag)


def _head_tc(p1, p2, p3, bng, bnb, wl1, bl1, wl2, bl2, eps):
    nb = p1.shape[0]
    cdim = wl2.shape[1]
    c0 = 1.0 / math.sqrt(1.0 + eps)

    def body(p1_ref, p2_ref, p3_ref, g_ref, b_ref, w1_ref, b1_ref,
             w2_ref, b2_ref, o_ref):
        z = jnp.concatenate([p1_ref[...], p2_ref[...], p3_ref[...]], axis=1)
        z = g_ref[...] * z * c0 + b_ref[...]
        z = jnp.maximum(
            jnp.dot(z, w1_ref[...], preferred_element_type=jnp.float32)
            + b1_ref[...], 0.0)
        z = jnp.dot(z, w2_ref[...], preferred_element_type=jnp.float32) \
            + b2_ref[...]
        m = jnp.max(z, axis=-1, keepdims=True)
        e = jnp.exp(z - m)
        o_ref[...] = e / jnp.sum(e, axis=-1, keepdims=True)

    return pl.pallas_call(
        body,
        out_shape=jax.ShapeDtypeStruct((nb, cdim), jnp.float32),
    )(p1, p2, p3, bng, bnb, wl1, bl1, wl2, bl2)


def kernel(x, edge_attr, params, edge_index, batch, batch_size):
    n, f = x.shape
    e = edge_attr.shape[0]
    blocks = params['blocks']
    h = blocks[0]['W0'].shape[1]
    nseg = 64
    eps = 1e-5

    ew_per_w = e // _NW
    nchunk = _cdiv(_cdiv(ew_per_w, _CH), 8) * 8
    ewp = nchunk * _CH
    pad = ewp - ew_per_w
    npad = _NS * _ZR * _cdiv(n, _NS * _ZR)

    src = edge_index[0].reshape(_NW, ew_per_w)
    dst = edge_index[1].reshape(_NW, ew_per_w)
    eww = edge_attr.reshape(_NW, ew_per_w)
    if pad:
        pad_src = jnp.broadcast_to(
            jnp.arange(pad, dtype=jnp.int32) % jnp.int32(n), (_NW, pad))
        pad_dst = jnp.broadcast_to(
            jnp.int32(n) + jnp.arange(pad, dtype=jnp.int32)
            % jnp.int32(npad - n), (_NW, pad))
        src = jnp.concatenate([src, pad_src], axis=1)
        dst = jnp.concatenate([dst, pad_dst], axis=1)
        eww = jnp.concatenate(
            [eww, jnp.zeros((_NW, pad), jnp.float32)], axis=1)
    src_p = src.reshape(_NW * nchunk, _CH)
    dst_p = dst.reshape(_NW * nchunk, _CH)
    ew_p = eww.reshape(_NW * nchunk, _CH)

    spmm = _make_spmm(n, h, nchunk, npad)
    degk = _make_deg(nchunk, npad)

    deg_flat = degk(dst_p, ew_p)
    dp = deg_flat.reshape(_NC, npad)[:, :n]
    dinv = _dinv_tc(dp).reshape(n, 1)

    batch_col = batch.reshape(n, 1)

    zh = jnp.zeros((h, h), jnp.float32)
    zb = jnp.zeros((h,), jnp.float32)
    b_st, wja_st, wjb_st, bj_st, wn_st, fl_st = [], [], [], [], [], []
    for bi in range(3):
        p = blocks[bi]
        wn_last = blocks[bi + 1]['W0'] if bi < 2 else zh
        b_st += [p['b0'], p['b1']]
        wja_st += [zh, p['Wj'][:h]]
        wjb_st += [zh, p['Wj'][h:]]
        bj_st += [zb, p['bj']]
        wn_st += [p['W1'], wn_last]
        fl_st += [0.0, 1.0]
    b_st = jnp.stack(b_st).reshape(6, 1, h)
    wja_st = jnp.stack(wja_st)
    wjb_st = jnp.stack(wjb_st)
    bj_st = jnp.stack(bj_st).reshape(6, 1, h)
    wn_st = jnp.stack(wn_st)
    fl_st = jnp.asarray(fl_st, jnp.float32).reshape(6, 1, 1)

    y0 = _mm_scale_tc(x, blocks[0]['W0'], dinv)

    def take(stack, i):
        return lax.dynamic_index_in_dim(stack, i, 0, keepdims=False)

    def loop_body(i, carry):
        y, hprev, pooled = carry
        accp = spmm(y, src_p, dst_p, ew_p)
        h1, ynext, pooled_i = _conv_step_tc(
            accp, y, hprev, dinv, batch_col,
            take(b_st, i), take(wja_st, i), take(wjb_st, i),
            take(bj_st, i), take(wn_st, i), take(fl_st, i), nseg)
        pooled = lax.dynamic_update_slice(
            pooled, pooled_i[None], (i // 2, 0, 0))
        return (ynext, h1, pooled)

    ub = lax.optimization_barrier(jnp.int32(6))
    pooled0 = jnp.zeros((3, nseg, h), jnp.float32)
    _, _, pooled_all = lax.fori_loop(0, ub, loop_body, (y0, y0, pooled0))

    out = _head_tc(pooled_all[0], pooled_all[1], pooled_all[2],
                   params['bn_g'].reshape(1, 3 * h),
                   params['bn_b'].reshape(1, 3 * h),
                   params['Wl1'], params['bl1'].reshape(1, h),
                   params['Wl2'], params['bl2'].reshape(1, 16),
                   eps)
    return out
